# baseline (speedup 1.0000x reference)
import functools

import jax
import jax.numpy as jnp
from jax import lax
from jax.experimental import pallas as pl
from jax.experimental.pallas import tpu as pltpu
from jax.experimental.pallas import tpu_sc as plsc

_NC = 2
_NS = 16
_NW = _NC * _NS
_CHUNK = 128
_NBUF = 4
_NPOS = 2
_LANES = 16


def _emb_body(nchunk, bsz, seq_len, dim, ids_hbm, table_hbm, pos_hbm, out_hbm,
              idx_v, buf0, buf1, buf2, buf3, pp0, pp1, pos_sh,
              gs0, gs1, gs2, gs3, os0, os1, os2, os3, ps0, ps1):
    c = lax.axis_index("c")
    s = lax.axis_index("s")
    wid = s * _NC + c
    base = wid * nchunk * _CHUNK
    seqs_per_w = (nchunk * _CHUNK) // seq_len
    chunks_per_seq = seq_len // _CHUNK

    pltpu.sync_copy(ids_hbm.at[pl.ds(wid * seqs_per_w, seqs_per_w)], idx_v)

    def idx_slice(cidx):
        return idx_v.at[cidx // chunks_per_seq,
                        pl.ds((cidx % chunks_per_seq) * _CHUNK, _CHUNK)]

    bufs = (buf0, buf1, buf2, buf3)
    gsems = (gs0, gs1, gs2, gs3)
    osems = (os0, os1, os2, os3)
    pbufs = (pp0, pp1)
    psems = (ps0, ps1)

    gathers = [None] * _NBUF
    out_copies = [None] * _NBUF
    pos_pulls = [None] * _NPOS

    for c0 in range(min(_NBUF - 1, nchunk)):
        gathers[c0] = pltpu.async_copy(
            table_hbm.at[idx_slice(c0)], bufs[c0], gsems[c0])

    @pl.when(s == 0)
    def _():
        pltpu.sync_copy(pos_hbm, pos_sh)

    plsc.subcore_barrier()

    def start_pos_pull(cidx, q):
        sl = pos_sh.at[pl.ds(((cidx * _CHUNK) % seq_len), _CHUNK)]
        return pltpu.async_copy(sl, pbufs[q], psems[q])

    for c0 in range(min(_NPOS, nchunk)):
        pos_pulls[c0] = start_pos_pull(c0, c0)

    for cidx in range(nchunk):
        p = cidx % _NBUF
        pq = cidx % _NPOS
        buf = bufs[p]
        gathers[p].wait()
        nxt = cidx + _NBUF - 1
        if nxt < nchunk:
            q = nxt % _NBUF
            if out_copies[q] is not None:
                out_copies[q].wait()
                out_copies[q] = None
            gathers[q] = pltpu.async_copy(
                table_hbm.at[idx_slice(nxt)], bufs[q], gsems[q])

        pos_pulls[pq].wait()
        pos_v = pbufs[pq]

        def add_rows(i, carry, buf=buf, pos_v=pos_v):
            for u in range(4):
                r = i * 4 + u
                for d in range(dim // _LANES):
                    v = pos_v[r, pl.ds(d * _LANES, _LANES)]
                    plsc.addupdate(buf.at[r, pl.ds(d * _LANES, _LANES)], v)
            return carry

        lax.fori_loop(0, _CHUNK // 4, add_rows, 0)

        if cidx + _NPOS < nchunk:
            pos_pulls[pq] = start_pos_pull(cidx + _NPOS, pq)

        out_copies[p] = pltpu.async_copy(
            buf, out_hbm.at[pl.ds(base + cidx * _CHUNK, _CHUNK)], osems[p])

    for cp in out_copies:
        if cp is not None:
            cp.wait()


def kernel(input_ids, char_table, pos_table):
    bsz, seq_len = input_ids.shape
    vocab, dim = char_table.shape
    total = bsz * seq_len
    rows_per_w = total // _NW
    nchunk = rows_per_w // _CHUNK

    mesh = plsc.VectorSubcoreMesh(core_axis_name="c", subcore_axis_name="s")
    body = functools.partial(_emb_body, nchunk, bsz, seq_len, dim)
    out = pl.kernel(
        body,
        out_type=jax.ShapeDtypeStruct((total, dim), jnp.float32),
        mesh=mesh,
        scratch_types=[
            pltpu.VMEM((total // _NW // seq_len, seq_len), jnp.int32),
            pltpu.VMEM((_CHUNK, dim), jnp.float32),
            pltpu.VMEM((_CHUNK, dim), jnp.float32),
            pltpu.VMEM((_CHUNK, dim), jnp.float32),
            pltpu.VMEM((_CHUNK, dim), jnp.float32),
            pltpu.VMEM((_CHUNK, dim), jnp.float32),
            pltpu.VMEM((_CHUNK, dim), jnp.float32),
            pltpu.VMEM_SHARED((seq_len, dim), jnp.float32),
            pltpu.SemaphoreType.DMA,
            pltpu.SemaphoreType.DMA,
            pltpu.SemaphoreType.DMA,
            pltpu.SemaphoreType.DMA,
            pltpu.SemaphoreType.DMA,
            pltpu.SemaphoreType.DMA,
            pltpu.SemaphoreType.DMA,
            pltpu.SemaphoreType.DMA,
            pltpu.SemaphoreType.DMA,
            pltpu.SemaphoreType.DMA,
        ],
    )(input_ids, char_table, pos_table)
    return out.reshape(bsz, seq_len, dim)


# speedup vs baseline: 1.0044x; 1.0044x over previous
import functools

import jax
import jax.numpy as jnp
from jax import lax
from jax.experimental import pallas as pl
from jax.experimental.pallas import tpu as pltpu
from jax.experimental.pallas import tpu_sc as plsc

_NC = 2
_NS = 16
_NW = _NC * _NS
_CHUNK = 128
_NBUF = 4
_NPOS = 2
_LANES = 16


def _emb_body(nchunk, bsz, seq_len, dim, ids_hbm, table_hbm, pos_hbm, out_hbm,
              idx_v, buf0, buf1, buf2, buf3, pp0, pp1, pos_sh,
              gs0, gs1, gs2, gs3, os0, os1, os2, os3, ps0, ps1):
    c = lax.axis_index("c")
    s = lax.axis_index("s")
    wid = s * _NC + c
    base = wid * nchunk * _CHUNK
    seqs_per_w = (nchunk * _CHUNK) // seq_len
    chunks_per_seq = seq_len // _CHUNK

    pltpu.sync_copy(ids_hbm.at[pl.ds(wid * seqs_per_w, seqs_per_w)], idx_v)

    def idx_slice(cidx):
        return idx_v.at[cidx // chunks_per_seq,
                        pl.ds((cidx % chunks_per_seq) * _CHUNK, _CHUNK)]

    bufs = (buf0, buf1, buf2, buf3)
    gsems = (gs0, gs1, gs2, gs3)
    osems = (os0, os1, os2, os3)
    pbufs = (pp0, pp1)
    psems = (ps0, ps1)

    gathers = [None] * _NBUF
    out_copies = [None] * _NBUF
    pos_pulls = [None] * _NPOS

    for c0 in range(min(_NBUF - 1, nchunk)):
        gathers[c0] = pltpu.async_copy(
            table_hbm.at[idx_slice(c0)], bufs[c0], gsems[c0])

    rows_per_stager = seq_len // _NS
    pltpu.sync_copy(pos_hbm.at[pl.ds(s * rows_per_stager, rows_per_stager)],
                    pos_sh.at[pl.ds(s * rows_per_stager, rows_per_stager)])

    plsc.subcore_barrier()

    def start_pos_pull(cidx, q):
        sl = pos_sh.at[pl.ds(((cidx * _CHUNK) % seq_len), _CHUNK)]
        return pltpu.async_copy(sl, pbufs[q], psems[q])

    for c0 in range(min(_NPOS, nchunk)):
        pos_pulls[c0] = start_pos_pull(c0, c0)

    for cidx in range(nchunk):
        p = cidx % _NBUF
        pq = cidx % _NPOS
        buf = bufs[p]
        gathers[p].wait()
        nxt = cidx + _NBUF - 1
        if nxt < nchunk:
            q = nxt % _NBUF
            if out_copies[q] is not None:
                out_copies[q].wait()
                out_copies[q] = None
            gathers[q] = pltpu.async_copy(
                table_hbm.at[idx_slice(nxt)], bufs[q], gsems[q])

        pos_pulls[pq].wait()
        pos_v = pbufs[pq]

        def add_rows(i, carry, buf=buf, pos_v=pos_v):
            for u in range(4):
                r = i * 4 + u
                for d in range(dim // _LANES):
                    v = pos_v[r, pl.ds(d * _LANES, _LANES)]
                    plsc.addupdate(buf.at[r, pl.ds(d * _LANES, _LANES)], v)
            return carry

        lax.fori_loop(0, _CHUNK // 4, add_rows, 0)

        if cidx + _NPOS < nchunk:
            pos_pulls[pq] = start_pos_pull(cidx + _NPOS, pq)

        out_copies[p] = pltpu.async_copy(
            buf, out_hbm.at[pl.ds(base + cidx * _CHUNK, _CHUNK)], osems[p])

    for cp in out_copies:
        if cp is not None:
            cp.wait()


def kernel(input_ids, char_table, pos_table):
    bsz, seq_len = input_ids.shape
    vocab, dim = char_table.shape
    total = bsz * seq_len
    rows_per_w = total // _NW
    nchunk = rows_per_w // _CHUNK

    mesh = plsc.VectorSubcoreMesh(core_axis_name="c", subcore_axis_name="s")
    body = functools.partial(_emb_body, nchunk, bsz, seq_len, dim)
    out = pl.kernel(
        body,
        out_type=jax.ShapeDtypeStruct((total, dim), jnp.float32),
        mesh=mesh,
        scratch_types=[
            pltpu.VMEM((total // _NW // seq_len, seq_len), jnp.int32),
            pltpu.VMEM((_CHUNK, dim), jnp.float32),
            pltpu.VMEM((_CHUNK, dim), jnp.float32),
            pltpu.VMEM((_CHUNK, dim), jnp.float32),
            pltpu.VMEM((_CHUNK, dim), jnp.float32),
            pltpu.VMEM((_CHUNK, dim), jnp.float32),
            pltpu.VMEM((_CHUNK, dim), jnp.float32),
            pltpu.VMEM_SHARED((seq_len, dim), jnp.float32),
            pltpu.SemaphoreType.DMA,
            pltpu.SemaphoreType.DMA,
            pltpu.SemaphoreType.DMA,
            pltpu.SemaphoreType.DMA,
            pltpu.SemaphoreType.DMA,
            pltpu.SemaphoreType.DMA,
            pltpu.SemaphoreType.DMA,
            pltpu.SemaphoreType.DMA,
            pltpu.SemaphoreType.DMA,
            pltpu.SemaphoreType.DMA,
        ],
    )(input_ids, char_table, pos_table)
    return out.reshape(bsz, seq_len, dim)
